# baseline (speedup 1.0000x reference)
import functools

import jax
import jax.numpy as jnp
from jax import lax
from jax.experimental import pallas as pl
from jax.experimental.pallas import tpu as pltpu
from jax.experimental.pallas import tpu_sc as plsc

N = 10000
E = 320000
G = 128
K = 64
NP = 10240
NW = 32
NCH = 80
C = 128
RPS = NP // 16

_ZR = {16: 128, 32: 128, 64: 64}

NEG_INF = float("-inf")


def _prop_round(F, zr, cid, sid, table_hbm, out_hbm,
                src_v, dst_v, rows_v, zbuf_v, gsem, ssem, acc_sh):
    for t in range(RPS // zr):
        pltpu.sync_copy(zbuf_v, acc_sh.at[pl.ds(sid * RPS + t * zr, zr)])
    plsc.subcore_barrier()

    def g_start(j, b):
        pltpu.make_async_copy(table_hbm.at[src_v.at[j]],
                              rows_v.at[b], gsem[b]).start()

    def g_wait(b):
        pltpu.make_async_copy(table_hbm.at[src_v.at[0]],
                              rows_v.at[b], gsem[b]).wait()

    def s_start(j, b):
        pltpu.async_copy(rows_v.at[b], acc_sh.at[dst_v.at[j]],
                         ssem[b], add=True)

    def s_wait(b):
        pltpu.make_async_copy(rows_v.at[b], acc_sh.at[dst_v.at[0]],
                              ssem[b]).wait()

    g_start(0, 0)
    g_start(1, 1)

    def body(jo, carry):
        for b in range(5):
            j = jo * 5 + b
            g_wait(b)
            s_start(j, b)

            @pl.when(j >= 3)
            def _():
                s_wait((b + 2) % 5)
            g_start(jnp.minimum(j + 2, NCH - 1), (b + 2) % 5)
        return carry

    lax.fori_loop(0, NCH // 5, body, 0)
    g_wait(0)
    g_wait(1)
    for b in range(2, 5):
        s_wait(b)
    plsc.subcore_barrier()

    pltpu.sync_copy(acc_sh.at[pl.ds(sid * RPS, RPS)],
                    out_hbm.at[cid, pl.ds(sid * RPS, RPS)])
    plsc.subcore_barrier()


def _make_propagate(F, nt=1):
    zr = _ZR[F]
    mesh = plsc.VectorSubcoreMesh(core_axis_name="c", subcore_axis_name="s")

    @functools.partial(
        pl.kernel,
        out_type=[jax.ShapeDtypeStruct((2, NP, F), jnp.float32)] * nt,
        mesh=mesh,
        compiler_params=pltpu.CompilerParams(use_tc_tiling_on_sc=False),
        scratch_types=[
            pltpu.VMEM((NCH, C), jnp.int32),
            pltpu.VMEM((NCH, C), jnp.int32),
            pltpu.VMEM((5, C, F), jnp.float32),
            pltpu.VMEM((zr, F), jnp.float32),
            pltpu.VMEM_SHARED((NP, F), jnp.float32),
        ] + [pltpu.SemaphoreType.DMA] * 10,
    )
    def prop(*args):
        tables = args[:nt]
        srcb_hbm, dstb_hbm = args[nt], args[nt + 1]
        outs = args[nt + 2:2 * nt + 2]
        src_v, dst_v, rows_v, zbuf_v, acc_sh = args[2 * nt + 2:2 * nt + 7]
        sems = args[2 * nt + 7:]
        gsem = sems[:5]
        ssem = sems[5:]
        cid = lax.axis_index("c")
        sid = lax.axis_index("s")
        wid = sid * 2 + cid

        pltpu.sync_copy(srcb_hbm.at[wid], src_v)
        pltpu.sync_copy(dstb_hbm.at[wid], dst_v)

        for r in range(zr):
            for c in range(F // 16):
                zbuf_v[r, pl.ds(c * 16, 16)] = jnp.zeros((16,), jnp.float32)

        for r in range(nt):
            _prop_round(F, zr, cid, sid, tables[r], outs[r],
                        src_v, dst_v, rows_v, zbuf_v, gsem, ssem, acc_sh)

    return prop


def _make_deg():
    mesh = plsc.VectorSubcoreMesh(core_axis_name="c", subcore_axis_name="s")

    @functools.partial(
        pl.kernel,
        out_type=jax.ShapeDtypeStruct((2, NP, 8), jnp.float32),
        mesh=mesh,
        compiler_params=pltpu.CompilerParams(use_tc_tiling_on_sc=False),
        scratch_types=[
            pltpu.VMEM((NCH, C), jnp.int32),
            pltpu.VMEM((C, 8), jnp.float32),
            pltpu.VMEM_SHARED((NP, 8), jnp.float32),
        ],
    )
    def deg(dstb_hbm, ones8_hbm, zer8_hbm, out_hbm, dst_v, ones_v, acc_sh):
        cid = lax.axis_index("c")
        sid = lax.axis_index("s")
        wid = sid * 2 + cid
        pltpu.sync_copy(dstb_hbm.at[wid], dst_v)
        pltpu.sync_copy(ones8_hbm, ones_v)
        pltpu.sync_copy(zer8_hbm, acc_sh.at[pl.ds(sid * RPS, RPS)])
        plsc.subcore_barrier()

        def body(j, carry):
            pltpu.sync_copy(ones_v, acc_sh.at[dst_v.at[j]], add=True)
            return carry

        lax.fori_loop(0, NCH, body, 0)
        plsc.subcore_barrier()
        pltpu.sync_copy(acc_sh.at[pl.ds(sid * RPS, RPS)],
                        out_hbm.at[cid, pl.ds(sid * RPS, RPS)])

    return deg


def _make_gather_rows():
    mesh = plsc.VectorSubcoreMesh(core_axis_name="c", subcore_axis_name="s")

    @functools.partial(
        pl.kernel,
        out_type=jax.ShapeDtypeStruct((G * K, 256), jnp.float32),
        mesh=mesh,
        compiler_params=pltpu.CompilerParams(use_tc_tiling_on_sc=False),
        scratch_types=[
            pltpu.VMEM((2, 128), jnp.int32),
            pltpu.VMEM((128, 256), jnp.float32),
            pltpu.VMEM((128, 256), jnp.float32),
            pltpu.SemaphoreType.DMA,
            pltpu.SemaphoreType.DMA,
            pltpu.SemaphoreType.DMA,
            pltpu.SemaphoreType.DMA,
        ],
    )
    def gat(table_hbm, sl_hbm, out_hbm, sl_v, rows0_v, rows1_v,
            sem0, sem1, sem2, sem3):
        cid = lax.axis_index("c")
        sid = lax.axis_index("s")
        wid = sid * 2 + cid
        base = wid * 256
        pltpu.sync_copy(sl_hbm.at[wid], sl_v)
        pltpu.make_async_copy(table_hbm.at[sl_v.at[0]], rows0_v, sem0).start()
        pltpu.make_async_copy(table_hbm.at[sl_v.at[1]], rows1_v, sem1).start()
        w0 = pltpu.make_async_copy(rows0_v, out_hbm.at[pl.ds(base, 128)], sem2)
        w1 = pltpu.make_async_copy(rows1_v, out_hbm.at[pl.ds(base + 128, 128)],
                                   sem3)
        pltpu.make_async_copy(table_hbm.at[sl_v.at[0]], rows0_v, sem0).wait()
        w0.start()
        pltpu.make_async_copy(table_hbm.at[sl_v.at[1]], rows1_v, sem1).wait()
        w1.start()
        w0.wait()
        w1.wait()

    return gat


def _dinv_of(degp_ref):
    deg = degp_ref[0, :, 0:1] + degp_ref[1, :, 0:1] + 1.0
    return lax.rsqrt(deg)


def _pre_body(x_ref, degp_ref, w_ref, hw_ref, hwsa_ref, hwsb_ref):
    dinv = _dinv_of(degp_ref)
    hw = jnp.dot(x_ref[...], w_ref[...], preferred_element_type=jnp.float32)
    hw_ref[...] = hw
    hws = dinv * hw
    hwsa_ref[...] = hws[:, :64]
    hwsb_ref[...] = hws[:, 64:]


def _combine_body(nblk, with_next, nagg, *refs):
    aggp_refs = refs[:nagg]
    hw_ref, degp_ref, b_ref = refs[nagg:nagg + 3]
    rest = refs[nagg + 3:]
    if with_next:
        w_ref, x_ref, hwn_ref, hwns_ref = rest
    else:
        (x_ref,) = rest
    i = pl.program_id(0)
    dinv = _dinv_of(degp_ref)
    agg = jnp.concatenate([a[0] + a[1] for a in aggp_refs], axis=1) \
        if nagg > 1 else (aggp_refs[0][0] + aggp_refs[0][1])
    xc = jnp.tanh(dinv * agg + (dinv * dinv) * hw_ref[...] + b_ref[...])
    row = i * nblk + lax.broadcasted_iota(jnp.int32, (nblk, 1), 0)
    xc = jnp.where(row < N, xc, 0.0)
    x_ref[...] = xc
    if with_next:
        hwn = jnp.dot(xc, w_ref[...], preferred_element_type=jnp.float32)
        hwn_ref[...] = hwn
        hwns_ref[...] = dinv * hwn


def _tc_pre(xp, degp, w1):
    nblk = 1024
    grid = NP // nblk
    fo = w1.shape[1]
    return pl.pallas_call(
        _pre_body,
        grid=(grid,),
        in_specs=[
            pl.BlockSpec((nblk, xp.shape[1]), lambda i: (i, 0)),
            pl.BlockSpec((2, nblk, 8), lambda i: (0, i, 0)),
            pl.BlockSpec(w1.shape, lambda i: (0, 0)),
        ],
        out_specs=[
            pl.BlockSpec((nblk, fo), lambda i: (i, 0)),
            pl.BlockSpec((nblk, fo // 2), lambda i: (i, 0)),
            pl.BlockSpec((nblk, fo // 2), lambda i: (i, 0)),
        ],
        out_shape=[
            jax.ShapeDtypeStruct((NP, fo), jnp.float32),
            jax.ShapeDtypeStruct((NP, fo // 2), jnp.float32),
            jax.ShapeDtypeStruct((NP, fo // 2), jnp.float32),
        ],
    )(xp, degp, w1)


def _tc_combine(aggps, hw, degp, b, w_next):
    nblk = 1024
    grid = NP // nblk
    fi = hw.shape[1]
    b2 = b.reshape(1, fi)
    with_next = w_next is not None
    in_specs = [
        pl.BlockSpec((2, nblk, a.shape[2]), lambda i: (0, i, 0)) for a in aggps
    ] + [
        pl.BlockSpec((nblk, fi), lambda i: (i, 0)),
        pl.BlockSpec((2, nblk, 8), lambda i: (0, i, 0)),
        pl.BlockSpec((1, fi), lambda i: (0, 0)),
    ]
    out_specs = [pl.BlockSpec((nblk, fi), lambda i: (i, 0))]
    out_shape = [jax.ShapeDtypeStruct((NP, fi), jnp.float32)]
    args = list(aggps) + [hw, degp, b2]
    if with_next:
        fo = w_next.shape[1]
        in_specs.append(pl.BlockSpec(w_next.shape, lambda i: (0, 0)))
        out_specs += [pl.BlockSpec((nblk, fo), lambda i: (i, 0))] * 2
        out_shape += [jax.ShapeDtypeStruct((NP, fo), jnp.float32)] * 2
        args.append(w_next)
    return pl.pallas_call(
        functools.partial(_combine_body, nblk, with_next, len(aggps)),
        grid=(grid,),
        in_specs=in_specs,
        out_specs=out_specs,
        out_shape=out_shape,
    )(*args)


def _topk_body(x4_ref, bat_ref, out_ref):
    score = x4_ref[:, 31:32]
    b = bat_ref[...]
    gcol = lax.broadcasted_iota(jnp.int32, (NP, G), 1)
    rowid = lax.broadcasted_iota(jnp.int32, (NP, G), 0)
    s0 = jnp.where(b == gcol, score, NEG_INF)

    def body(t, s):
        m = jnp.max(s, axis=0, keepdims=True)
        sel = jnp.where(s == m, rowid, jnp.int32(2**30))
        idx = jnp.min(sel, axis=0, keepdims=True)
        idxf = jnp.where(m == NEG_INF, jnp.int32(N), idx)
        out_ref[pl.ds(t, 1), :] = idxf
        return jnp.where(rowid == idx, NEG_INF, s)

    lax.fori_loop(0, K, body, s0)


def _tc_topk(x4, bat2d):
    return pl.pallas_call(
        _topk_body,
        out_shape=jax.ShapeDtypeStruct((K, G), jnp.int32),
    )(x4, bat2d)


def _conv5_body(p_ref, c5wt_ref, c5b_ref, z1_ref):
    z1_ref[...] = jnp.maximum(
        jnp.dot(p_ref[...], c5wt_ref[...], preferred_element_type=jnp.float32)
        + c5b_ref[...], 0.0)


def _pool_body(zr_ref, zp_ref):
    zr = zr_ref[...]
    zp_ref[...] = jnp.maximum(zr[:, :16], zr[:, 16:])


def _conv6_body(zts_ref, c6wt_ref, c6b_ref, z6_ref):
    acc = c6b_ref[...]
    for dt in range(5):
        acc = acc + jnp.dot(zts_ref[dt], c6wt_ref[dt],
                            preferred_element_type=jnp.float32)
    z6_ref[...] = jnp.maximum(acc, 0.0)


def _fc_body(zf_ref, fc1wp_ref, fc1b_ref, fc2w_ref, fc2b_ref, out_ref):
    h = jnp.maximum(
        jnp.dot(zf_ref[...], fc1wp_ref[...],
                preferred_element_type=jnp.float32) + fc1b_ref[...], 0.0)
    o = jnp.dot(h, fc2w_ref[...], preferred_element_type=jnp.float32) \
        + fc2b_ref[...]
    m = jnp.max(o, axis=1, keepdims=True)
    lse = m + jnp.log(jnp.sum(jnp.exp(o - m), axis=1, keepdims=True))
    out_ref[...] = o - lse


def _tc_tail(p, c5wt, c5b2, c6wt, c6b2, fc1wp, fc1b2, fc2w, fc2b2):
    z1 = pl.pallas_call(
        _conv5_body,
        out_shape=jax.ShapeDtypeStruct((G * K, 16), jnp.float32),
    )(p, c5wt, c5b2)
    zr = z1.reshape(G * 32, 32)
    zp = pl.pallas_call(
        _pool_body,
        out_shape=jax.ShapeDtypeStruct((G * 32, 16), jnp.float32),
    )(zr)
    zp3 = zp.reshape(G, 32, 16)
    zts = jnp.stack([zp3[:, dt:dt + 28, :].reshape(G * 28, 16)
                     for dt in range(5)])
    z6 = pl.pallas_call(
        _conv6_body,
        out_shape=jax.ShapeDtypeStruct((G * 28, 32), jnp.float32),
    )(zts, c6wt, c6b2)
    zf = z6.reshape(G, 896)
    return pl.pallas_call(
        _fc_body,
        out_shape=jax.ShapeDtypeStruct((G, 10), jnp.float32),
    )(zf, fc1wp, fc1b2, fc2w, fc2b2)


def kernel(x, edge_index, batch, W1, b1, W2, b2, W3, b3, W4, b4,
           c5w, c5b, c6w, c6b, fc1w, fc1b, fc2w, fc2b):
    src = edge_index[0]
    dst = edge_index[1]
    pad_e = NW * NCH * C - E
    srcb = jnp.concatenate(
        [src, jnp.full((pad_e,), N, jnp.int32)]).reshape(NW, NCH, C)
    dstb = jnp.concatenate(
        [dst, jnp.full((pad_e,), N, jnp.int32)]).reshape(NW, NCH, C)

    xp = jnp.pad(x, ((0, NP - N), (0, 0)))
    bat2d = jnp.pad(batch, (0, NP - N), constant_values=G).reshape(NP, 1)
    prop64x2 = _make_propagate(64, nt=2)
    prop64 = _make_propagate(64)
    prop32 = _make_propagate(32)

    ones8 = jnp.ones((C, 8), jnp.float32)
    zer8 = jnp.zeros((RPS, 8), jnp.float32)
    degp = _make_deg()(dstb, ones8, zer8)

    hw1, hw1sa, hw1sb = _tc_pre(xp, degp, W1)
    agg1a, agg1b = prop64x2(hw1sa, hw1sb, srcb, dstb)
    x1, hw2, hw2s = _tc_combine([agg1a, agg1b], hw1, degp, b1, W2)
    (agg2,) = prop64(hw2s, srcb, dstb)
    x2, hw3, hw3s = _tc_combine([agg2], hw2, degp, b2, W3)
    (agg3,) = prop32(hw3s, srcb, dstb)
    x3, hw4, hw4s = _tc_combine([agg3], hw3, degp, b3, W4)
    (agg4,) = prop32(hw4s, srcb, dstb)
    (x4,) = _tc_combine([agg4], hw4, degp, b4, None)

    xc = jnp.concatenate([x1, x2, x3, x4], axis=1)

    slots = _tc_topk(x4, bat2d)
    sl = slots.T.reshape(NW, 2, 128)

    p = _make_gather_rows()(xc, sl)

    c5wt = jnp.transpose(c5w[:, 0, :], (1, 0))
    c6wt = jnp.transpose(c6w, (2, 1, 0))
    fc1wp = fc1w.reshape(32, 28, 128).transpose(1, 0, 2).reshape(896, 128)
    return _tc_tail(p, c5wt, c5b.reshape(1, 16), c6wt, c6b.reshape(1, 32),
                    fc1wp, fc1b.reshape(1, 128), fc2w, fc2b.reshape(1, 10))


# speedup vs baseline: 1.0627x; 1.0627x over previous
import functools

import jax
import jax.numpy as jnp
from jax import lax
from jax.experimental import pallas as pl
from jax.experimental.pallas import tpu as pltpu
from jax.experimental.pallas import tpu_sc as plsc

N = 10000
E = 320000
G = 128
K = 64
NP = 10240
NW = 32
NCH = 80
C = 128
RPS = NP // 16

_ZR = {16: 128, 32: 128, 64: 64}

NEG_INF = float("-inf")


def _make_propagate(F):
    zr = _ZR[F]
    mesh = plsc.VectorSubcoreMesh(core_axis_name="c", subcore_axis_name="s")

    @functools.partial(
        pl.kernel,
        out_type=jax.ShapeDtypeStruct((2, NP, F), jnp.float32),
        mesh=mesh,
        compiler_params=pltpu.CompilerParams(use_tc_tiling_on_sc=False),
        scratch_types=[
            pltpu.VMEM((NCH, C), jnp.int32),
            pltpu.VMEM((NCH, C), jnp.int32),
            pltpu.VMEM((5, C, F), jnp.float32),
            pltpu.VMEM((zr, F), jnp.float32),
            pltpu.VMEM_SHARED((NP, F), jnp.float32),
        ] + [pltpu.SemaphoreType.DMA] * 10,
    )
    def prop(table_hbm, srcb_hbm, dstb_hbm, out_hbm,
             src_v, dst_v, rows_v, zbuf_v, acc_sh, *sems):
        gsem = sems[:5]
        ssem = sems[5:]
        cid = lax.axis_index("c")
        sid = lax.axis_index("s")
        wid = sid * 2 + cid

        pltpu.sync_copy(srcb_hbm.at[wid], src_v)
        pltpu.sync_copy(dstb_hbm.at[wid], dst_v)

        for r in range(zr):
            for c in range(F // 16):
                zbuf_v[r, pl.ds(c * 16, 16)] = jnp.zeros((16,), jnp.float32)
        for t in range(RPS // zr):
            pltpu.sync_copy(zbuf_v, acc_sh.at[pl.ds(sid * RPS + t * zr, zr)])
        plsc.subcore_barrier()

        def g_start(j, b):
            pltpu.make_async_copy(table_hbm.at[src_v.at[j]],
                                  rows_v.at[b], gsem[b]).start()

        def g_wait(b):
            pltpu.make_async_copy(table_hbm.at[src_v.at[0]],
                                  rows_v.at[b], gsem[b]).wait()

        def s_start(j, b):
            pltpu.async_copy(rows_v.at[b], acc_sh.at[dst_v.at[j]],
                             ssem[b], add=True)

        def s_wait(b):
            pltpu.make_async_copy(rows_v.at[b], acc_sh.at[dst_v.at[0]],
                                  ssem[b]).wait()

        g_start(0, 0)
        g_start(1, 1)

        def body(jo, carry):
            for b in range(5):
                j = jo * 5 + b
                g_wait(b)
                s_start(j, b)

                @pl.when(j >= 3)
                def _():
                    s_wait((b + 2) % 5)
                g_start(jnp.minimum(j + 2, NCH - 1), (b + 2) % 5)
            return carry

        lax.fori_loop(0, NCH // 5, body, 0)
        g_wait(0)
        g_wait(1)
        for b in range(2, 5):
            s_wait(b)
        plsc.subcore_barrier()

        pltpu.sync_copy(acc_sh.at[pl.ds(sid * RPS, RPS)],
                        out_hbm.at[cid, pl.ds(sid * RPS, RPS)])

    return prop


def _make_deg():
    mesh = plsc.VectorSubcoreMesh(core_axis_name="c", subcore_axis_name="s")

    @functools.partial(
        pl.kernel,
        out_type=jax.ShapeDtypeStruct((2, NP, 8), jnp.float32),
        mesh=mesh,
        compiler_params=pltpu.CompilerParams(use_tc_tiling_on_sc=False),
        scratch_types=[
            pltpu.VMEM((NCH, C), jnp.int32),
            pltpu.VMEM((C, 8), jnp.float32),
            pltpu.VMEM_SHARED((NP, 8), jnp.float32),
        ],
    )
    def deg(dstb_hbm, ones8_hbm, zer8_hbm, out_hbm, dst_v, ones_v, acc_sh):
        cid = lax.axis_index("c")
        sid = lax.axis_index("s")
        wid = sid * 2 + cid
        pltpu.sync_copy(dstb_hbm.at[wid], dst_v)
        pltpu.sync_copy(ones8_hbm, ones_v)
        pltpu.sync_copy(zer8_hbm, acc_sh.at[pl.ds(sid * RPS, RPS)])
        plsc.subcore_barrier()

        def body(j, carry):
            pltpu.sync_copy(ones_v, acc_sh.at[dst_v.at[j]], add=True)
            return carry

        lax.fori_loop(0, NCH, body, 0)
        plsc.subcore_barrier()
        pltpu.sync_copy(acc_sh.at[pl.ds(sid * RPS, RPS)],
                        out_hbm.at[cid, pl.ds(sid * RPS, RPS)])

    return deg


def _make_gather_rows():
    mesh = plsc.VectorSubcoreMesh(core_axis_name="c", subcore_axis_name="s")

    @functools.partial(
        pl.kernel,
        out_type=jax.ShapeDtypeStruct((G * K, 256), jnp.float32),
        mesh=mesh,
        compiler_params=pltpu.CompilerParams(use_tc_tiling_on_sc=False),
        scratch_types=[
            pltpu.VMEM((2, 128), jnp.int32),
            pltpu.VMEM((128, 256), jnp.float32),
            pltpu.VMEM((128, 256), jnp.float32),
            pltpu.SemaphoreType.DMA,
            pltpu.SemaphoreType.DMA,
        ],
    )
    def gat(table_hbm, sl_hbm, out_hbm, sl_v, rows0_v, rows1_v, sem0, sem1):
        cid = lax.axis_index("c")
        sid = lax.axis_index("s")
        wid = sid * 2 + cid
        base = wid * 256
        pltpu.sync_copy(sl_hbm.at[wid], sl_v)
        pltpu.make_async_copy(table_hbm.at[sl_v.at[0]], rows0_v, sem0).start()
        pltpu.make_async_copy(table_hbm.at[sl_v.at[1]], rows1_v, sem1).start()
        pltpu.make_async_copy(table_hbm.at[sl_v.at[0]], rows0_v, sem0).wait()
        pltpu.sync_copy(rows0_v, out_hbm.at[pl.ds(base, 128)])
        pltpu.make_async_copy(table_hbm.at[sl_v.at[1]], rows1_v, sem1).wait()
        pltpu.sync_copy(rows1_v, out_hbm.at[pl.ds(base + 128, 128)])

    return gat


def _dinv_of(degp_ref):
    deg = degp_ref[0, :, 0:1] + degp_ref[1, :, 0:1] + 1.0
    return lax.rsqrt(deg)


def _pre_body(x_ref, degp_ref, w_ref, hw_ref, hwsa_ref, hwsb_ref):
    dinv = _dinv_of(degp_ref)
    hw = jnp.dot(x_ref[...], w_ref[...], preferred_element_type=jnp.float32)
    hw_ref[...] = hw
    hws = dinv * hw
    hwsa_ref[...] = hws[:, :64]
    hwsb_ref[...] = hws[:, 64:]


def _combine_body(nblk, with_next, nagg, *refs):
    aggp_refs = refs[:nagg]
    hw_ref, degp_ref, b_ref = refs[nagg:nagg + 3]
    rest = refs[nagg + 3:]
    if with_next:
        w_ref, x_ref, hwn_ref, hwns_ref = rest
    else:
        (x_ref,) = rest
    i = pl.program_id(0)
    dinv = _dinv_of(degp_ref)
    agg = jnp.concatenate([a[0] + a[1] for a in aggp_refs], axis=1) \
        if nagg > 1 else (aggp_refs[0][0] + aggp_refs[0][1])
    xc = jnp.tanh(dinv * agg + (dinv * dinv) * hw_ref[...] + b_ref[...])
    row = i * nblk + lax.broadcasted_iota(jnp.int32, (nblk, 1), 0)
    xc = jnp.where(row < N, xc, 0.0)
    x_ref[...] = xc
    if with_next:
        hwn = jnp.dot(xc, w_ref[...], preferred_element_type=jnp.float32)
        hwn_ref[...] = hwn
        hwns_ref[...] = dinv * hwn


def _tc_pre(xp, degp, w1):
    nblk = 1024
    grid = NP // nblk
    fo = w1.shape[1]
    return pl.pallas_call(
        _pre_body,
        grid=(grid,),
        in_specs=[
            pl.BlockSpec((nblk, xp.shape[1]), lambda i: (i, 0)),
            pl.BlockSpec((2, nblk, 8), lambda i: (0, i, 0)),
            pl.BlockSpec(w1.shape, lambda i: (0, 0)),
        ],
        out_specs=[
            pl.BlockSpec((nblk, fo), lambda i: (i, 0)),
            pl.BlockSpec((nblk, fo // 2), lambda i: (i, 0)),
            pl.BlockSpec((nblk, fo // 2), lambda i: (i, 0)),
        ],
        out_shape=[
            jax.ShapeDtypeStruct((NP, fo), jnp.float32),
            jax.ShapeDtypeStruct((NP, fo // 2), jnp.float32),
            jax.ShapeDtypeStruct((NP, fo // 2), jnp.float32),
        ],
    )(xp, degp, w1)


def _tc_combine(aggps, hw, degp, b, w_next):
    nblk = 1024
    grid = NP // nblk
    fi = hw.shape[1]
    b2 = b.reshape(1, fi)
    with_next = w_next is not None
    in_specs = [
        pl.BlockSpec((2, nblk, a.shape[2]), lambda i: (0, i, 0)) for a in aggps
    ] + [
        pl.BlockSpec((nblk, fi), lambda i: (i, 0)),
        pl.BlockSpec((2, nblk, 8), lambda i: (0, i, 0)),
        pl.BlockSpec((1, fi), lambda i: (0, 0)),
    ]
    out_specs = [pl.BlockSpec((nblk, fi), lambda i: (i, 0))]
    out_shape = [jax.ShapeDtypeStruct((NP, fi), jnp.float32)]
    args = list(aggps) + [hw, degp, b2]
    if with_next:
        fo = w_next.shape[1]
        in_specs.append(pl.BlockSpec(w_next.shape, lambda i: (0, 0)))
        out_specs += [pl.BlockSpec((nblk, fo), lambda i: (i, 0))] * 2
        out_shape += [jax.ShapeDtypeStruct((NP, fo), jnp.float32)] * 2
        args.append(w_next)
    return pl.pallas_call(
        functools.partial(_combine_body, nblk, with_next, len(aggps)),
        grid=(grid,),
        in_specs=in_specs,
        out_specs=out_specs,
        out_shape=out_shape,
    )(*args)


def _topk_body(x4_ref, bat_ref, out_ref):
    score = x4_ref[:, 31:32]
    b = bat_ref[...]
    gcol = lax.broadcasted_iota(jnp.int32, (NP, G), 1)
    rowid = lax.broadcasted_iota(jnp.int32, (NP, G), 0)
    s0 = jnp.where(b == gcol, score, NEG_INF)

    def body(t, s):
        m = jnp.max(s, axis=0, keepdims=True)
        sel = jnp.where(s == m, rowid, jnp.int32(2**30))
        idx = jnp.min(sel, axis=0, keepdims=True)
        idxf = jnp.where(m == NEG_INF, jnp.int32(N), idx)
        out_ref[pl.ds(t, 1), :] = idxf
        return jnp.where(rowid == idx, NEG_INF, s)

    lax.fori_loop(0, K, body, s0)


def _tc_topk(x4, bat2d):
    return pl.pallas_call(
        _topk_body,
        out_shape=jax.ShapeDtypeStruct((K, G), jnp.int32),
    )(x4, bat2d)


def _conv5_body(p_ref, c5wt_ref, c5b_ref, z1_ref):
    z1_ref[...] = jnp.maximum(
        jnp.dot(p_ref[...], c5wt_ref[...], preferred_element_type=jnp.float32)
        + c5b_ref[...], 0.0)


def _pool_body(zr_ref, zp_ref):
    zr = zr_ref[...]
    zp_ref[...] = jnp.maximum(zr[:, :16], zr[:, 16:])


def _conv6_body(zts_ref, c6wt_ref, c6b_ref, z6_ref):
    acc = c6b_ref[...]
    for dt in range(5):
        acc = acc + jnp.dot(zts_ref[dt], c6wt_ref[dt],
                            preferred_element_type=jnp.float32)
    z6_ref[...] = jnp.maximum(acc, 0.0)


def _fc_body(zf_ref, fc1wp_ref, fc1b_ref, fc2w_ref, fc2b_ref, out_ref):
    h = jnp.maximum(
        jnp.dot(zf_ref[...], fc1wp_ref[...],
                preferred_element_type=jnp.float32) + fc1b_ref[...], 0.0)
    o = jnp.dot(h, fc2w_ref[...], preferred_element_type=jnp.float32) \
        + fc2b_ref[...]
    m = jnp.max(o, axis=1, keepdims=True)
    lse = m + jnp.log(jnp.sum(jnp.exp(o - m), axis=1, keepdims=True))
    out_ref[...] = o - lse


def _tc_tail(p, c5wt, c5b2, c6wt, c6b2, fc1wp, fc1b2, fc2w, fc2b2):
    z1 = pl.pallas_call(
        _conv5_body,
        out_shape=jax.ShapeDtypeStruct((G * K, 16), jnp.float32),
    )(p, c5wt, c5b2)
    zr = z1.reshape(G * 32, 32)
    zp = pl.pallas_call(
        _pool_body,
        out_shape=jax.ShapeDtypeStruct((G * 32, 16), jnp.float32),
    )(zr)
    zp3 = zp.reshape(G, 32, 16)
    zts = jnp.stack([zp3[:, dt:dt + 28, :].reshape(G * 28, 16)
                     for dt in range(5)])
    z6 = pl.pallas_call(
        _conv6_body,
        out_shape=jax.ShapeDtypeStruct((G * 28, 32), jnp.float32),
    )(zts, c6wt, c6b2)
    zf = z6.reshape(G, 896)
    return pl.pallas_call(
        _fc_body,
        out_shape=jax.ShapeDtypeStruct((G, 10), jnp.float32),
    )(zf, fc1wp, fc1b2, fc2w, fc2b2)


def kernel(x, edge_index, batch, W1, b1, W2, b2, W3, b3, W4, b4,
           c5w, c5b, c6w, c6b, fc1w, fc1b, fc2w, fc2b):
    src = edge_index[0]
    dst = edge_index[1]
    pad_e = NW * NCH * C - E
    srcb = jnp.concatenate(
        [src, jnp.full((pad_e,), N, jnp.int32)]).reshape(NW, NCH, C)
    dstb = jnp.concatenate(
        [dst, jnp.full((pad_e,), N, jnp.int32)]).reshape(NW, NCH, C)

    xp = jnp.pad(x, ((0, NP - N), (0, 0)))
    bat2d = jnp.pad(batch, (0, NP - N), constant_values=G).reshape(NP, 1)
    prop64 = _make_propagate(64)
    prop32 = _make_propagate(32)

    ones8 = jnp.ones((C, 8), jnp.float32)
    zer8 = jnp.zeros((RPS, 8), jnp.float32)
    degp = _make_deg()(dstb, ones8, zer8)

    hw1, hw1sa, hw1sb = _tc_pre(xp, degp, W1)
    agg1a = prop64(hw1sa, srcb, dstb)
    agg1b = prop64(hw1sb, srcb, dstb)
    x1, hw2, hw2s = _tc_combine([agg1a, agg1b], hw1, degp, b1, W2)
    agg2 = prop64(hw2s, srcb, dstb)
    x2, hw3, hw3s = _tc_combine([agg2], hw2, degp, b2, W3)
    agg3 = prop32(hw3s, srcb, dstb)
    x3, hw4, hw4s = _tc_combine([agg3], hw3, degp, b3, W4)
    agg4 = prop32(hw4s, srcb, dstb)
    (x4,) = _tc_combine([agg4], hw4, degp, b4, None)

    xc = jnp.concatenate([x1, x2, x3, x4], axis=1)

    slots = _tc_topk(x4, bat2d)
    sl = slots.T.reshape(NW, 2, 128)

    p = _make_gather_rows()(xc, sl)

    c5wt = jnp.transpose(c5w[:, 0, :], (1, 0))
    c6wt = jnp.transpose(c6w, (2, 1, 0))
    fc1wp = fc1w.reshape(32, 28, 128).transpose(1, 0, 2).reshape(896, 128)
    return _tc_tail(p, c5wt, c5b.reshape(1, 16), c6wt, c6b.reshape(1, 32),
                    fc1wp, fc1b.reshape(1, 128), fc2w, fc2b.reshape(1, 10))
